# baseline (speedup 1.0000x reference)
import functools

import jax
import jax.numpy as jnp
from jax import lax
from jax.experimental import pallas as pl
from jax.experimental.pallas import tpu as pltpu
from jax.experimental.pallas import tpu_sc as plsc

BATCH = 4096
VOCAB = 1000000
DIM = 32
NNZ = 106496

NC = 2
NS = 16
NW = NC * NS
CHUNK = 128
PER_W = NNZ // NW
NCHUNK = PER_W // CHUNK
ROWS_PER_TILE = BATCH // NS

CB = 65536
Q = CB // 4
QSHIFT = Q.bit_length() - 1
NBLK = (VOCAB + CB - 1) // CB
VOCAB_PAD = NBLK * CB


def _tp_body(t_ref, o_ref):
    x = t_ref[...]
    x4 = jnp.concatenate([x[:, Q * j:Q * (j + 1)] for j in range(4)], axis=0)
    eye = jnp.eye(128, dtype=jnp.float32)
    o_ref[...] = jax.lax.dot(x4.T, eye, precision=jax.lax.Precision.DEFAULT)


def _transpose_pack(tab):
    tabT = tab.T
    return pl.pallas_call(
        _tp_body,
        grid=(NBLK,),
        in_specs=[pl.BlockSpec((32, CB), lambda c: (0, c))],
        out_specs=pl.BlockSpec((Q, 128), lambda c: (c, 0)),
        out_shape=jax.ShapeDtypeStruct((VOCAB_PAD // 4, 128), jnp.float32),
    )(tabT)


def _sc_body(idx_hbm, val_hbm, rid_hbm, tab_hbm, out_hbm,
             raw2, idx2, rid2, val2, rows_a, rows_b, w_a, w_b,
             acc_sh, sem_a, sem_b):
    cid = lax.axis_index("c")
    sid = lax.axis_index("s")
    wid = cid * NS + sid

    zeros = jnp.zeros((16,), jnp.float32)

    def _zero(j, _):
        rows_a[j, pl.ds(0, 16)] = zeros
        rows_a[j, pl.ds(16, 16)] = zeros
        return 0
    lax.fori_loop(0, CHUNK, _zero, 0)
    pltpu.sync_copy(rows_a, acc_sh.at[pl.ds(sid * ROWS_PER_TILE, CHUNK)])
    pltpu.sync_copy(rows_a, acc_sh.at[pl.ds(sid * ROWS_PER_TILE + CHUNK, CHUNK)])
    plsc.subcore_barrier()

    pltpu.sync_copy(idx_hbm.at[wid], raw2)
    pltpu.sync_copy(rid_hbm.at[wid], rid2)
    pltpu.sync_copy(val_hbm.at[wid], val2)

    def _remap_c(c, _):
        def _remap_g(g, _):
            raw = raw2[c, pl.ds(g * 16, 16)]
            idx2[c, pl.ds(g * 16, 16)] = (
                (raw & ~(CB - 1))
                + ((raw & (Q - 1)) << 2)
                + ((raw >> QSHIFT) & 3)
            )
            return 0
        lax.fori_loop(0, CHUNK // 16, _remap_g, 0)
        return 0
    lax.fori_loop(0, NCHUNK, _remap_c, 0)

    pltpu.async_copy(tab_hbm.at[idx2.at[0]], rows_a, sem_a)
    pltpu.async_copy(tab_hbm.at[idx2.at[1]], rows_b, sem_b)

    def _half(c, rows_v, w_v, sem):
        pltpu.make_async_copy(tab_hbm.at[idx2.at[c]], rows_v, sem).wait()

        def _scale(g, _):
            vvec = val2[c, pl.ds(g * 16, 16)]
            for t in range(16):
                j = g * 16 + t
                v = vvec[t]
                w_v[j, pl.ds(0, 16)] = rows_v[j, pl.ds(0, 16)] * v
                w_v[j, pl.ds(16, 16)] = rows_v[j, pl.ds(16, 16)] * v
            return 0
        lax.fori_loop(0, CHUNK // 16, _scale, 0)

        @pl.when(c + 2 < NCHUNK)
        def _():
            pltpu.async_copy(tab_hbm.at[idx2.at[c + 2]], rows_v, sem)

        pltpu.sync_copy(w_v, acc_sh.at[rid2.at[c]], add=True)

    def _pair(i, _):
        _half(i * 2, rows_a, w_a, sem_a)
        _half(i * 2 + 1, rows_b, w_b, sem_b)
        return 0
    lax.fori_loop(0, NCHUNK // 2, _pair, 0)
    plsc.subcore_barrier()

    pltpu.sync_copy(acc_sh.at[pl.ds(sid * ROWS_PER_TILE, ROWS_PER_TILE)],
                    out_hbm.at[cid, pl.ds(sid * ROWS_PER_TILE, ROWS_PER_TILE)])


def _sc_lookup(sp_indices, sp_values, sp_row_ids, tab_lin):
    mesh = plsc.VectorSubcoreMesh(core_axis_name="c", subcore_axis_name="s")
    idx_r = sp_indices.reshape(NW, NCHUNK, CHUNK)
    val_r = sp_values.reshape(NW, NCHUNK, CHUNK)
    rid_r = sp_row_ids.reshape(NW, NCHUNK, CHUNK)
    return pl.kernel(
        _sc_body,
        out_type=jax.ShapeDtypeStruct((NC, BATCH, DIM), jnp.float32),
        mesh=mesh,
        compiler_params=pltpu.CompilerParams(use_tc_tiling_on_sc=False),
        scratch_types=[
            pltpu.VMEM((NCHUNK, CHUNK), jnp.int32),
            pltpu.VMEM((NCHUNK, CHUNK), jnp.int32),
            pltpu.VMEM((NCHUNK, CHUNK), jnp.int32),
            pltpu.VMEM((NCHUNK, CHUNK), jnp.float32),
            pltpu.VMEM((CHUNK, DIM), jnp.float32),
            pltpu.VMEM((CHUNK, DIM), jnp.float32),
            pltpu.VMEM((CHUNK, DIM), jnp.float32),
            pltpu.VMEM((CHUNK, DIM), jnp.float32),
            pltpu.VMEM_SHARED((BATCH, DIM), jnp.float32),
            pltpu.SemaphoreType.DMA,
            pltpu.SemaphoreType.DMA,
        ],
    )(idx_r, val_r, rid_r, tab_lin)


def _add_body(p_ref, o_ref):
    o_ref[...] = (p_ref[0] + p_ref[1]).T


@jax.jit
def _run(sp_indices, sp_values, sp_row_ids, tab):
    packed = _transpose_pack(tab)
    tab_lin = packed.reshape(VOCAB_PAD, DIM)
    partials = _sc_lookup(sp_indices, sp_values, sp_row_ids, tab_lin)
    outT = pl.pallas_call(
        _add_body,
        out_shape=jax.ShapeDtypeStruct((DIM, BATCH), jnp.float32),
    )(partials)
    return outT.T


def kernel(sp_indices, sp_values, sp_row_ids, kernel):
    return _run(sp_indices, sp_values, sp_row_ids, kernel)


# speedup vs baseline: 1.0058x; 1.0058x over previous
import jax
import jax.numpy as jnp
from jax import lax
from jax.experimental import pallas as pl
from jax.experimental.pallas import tpu as pltpu
from jax.experimental.pallas import tpu_sc as plsc

BATCH = 4096
VOCAB = 1000000
DIM = 32
NNZ = 106496

NC = 2
NS = 16
NW = NC * NS
CHUNK = 128
PER_W = NNZ // NW
NCHUNK = PER_W // CHUNK
ROWS_PER_TILE = BATCH // NS

CB = 32768
Q = CB // 4
QSHIFT = Q.bit_length() - 1
NBLK = (VOCAB + CB - 1) // CB
VOCAB_PAD = NBLK * CB


def _tp_body(t_ref, o_ref):
    x = t_ref[...]
    x4 = jnp.concatenate([x[:, Q * j:Q * (j + 1)] for j in range(4)], axis=0)
    eye = jnp.eye(128, dtype=jnp.float32)
    o_ref[...] = jax.lax.dot(x4.T, eye, precision=jax.lax.Precision.DEFAULT)


def _transpose_pack(tab):
    tabT = tab.T
    return pl.pallas_call(
        _tp_body,
        grid=(NBLK,),
        in_specs=[pl.BlockSpec((32, CB), lambda c: (0, c))],
        out_specs=pl.BlockSpec((Q, 128), lambda c: (c, 0)),
        out_shape=jax.ShapeDtypeStruct((VOCAB_PAD // 4, 128), jnp.float32),
    )(tabT)


def _sc_body(idx_hbm, val_hbm, rid_hbm, tab_hbm, out_hbm,
             raw2, idx2, rid2, val2, rows_a, rows_b, w_a, w_b,
             acc_sh, sem_a, sem_b):
    cid = lax.axis_index("c")
    sid = lax.axis_index("s")
    wid = cid * NS + sid

    zeros = jnp.zeros((16,), jnp.float32)

    def _zero(j, _):
        rows_a[j, pl.ds(0, 16)] = zeros
        rows_a[j, pl.ds(16, 16)] = zeros
        return 0
    lax.fori_loop(0, CHUNK, _zero, 0)
    pltpu.sync_copy(rows_a, acc_sh.at[pl.ds(sid * ROWS_PER_TILE, CHUNK)])
    pltpu.sync_copy(rows_a, acc_sh.at[pl.ds(sid * ROWS_PER_TILE + CHUNK, CHUNK)])
    plsc.subcore_barrier()

    pltpu.sync_copy(idx_hbm.at[wid], raw2)
    pltpu.sync_copy(rid_hbm.at[wid], rid2)
    pltpu.sync_copy(val_hbm.at[wid], val2)

    def _remap_c(c, _):
        def _remap_g(g, _):
            raw = raw2[c, pl.ds(g * 16, 16)]
            idx2[c, pl.ds(g * 16, 16)] = (
                (raw & ~(CB - 1))
                + ((raw & (Q - 1)) << 2)
                + ((raw >> QSHIFT) & 3)
            )
            return 0
        lax.fori_loop(0, CHUNK // 16, _remap_g, 0)
        return 0
    lax.fori_loop(0, NCHUNK, _remap_c, 0)

    pltpu.async_copy(tab_hbm.at[idx2.at[0]], rows_a, sem_a)
    pltpu.async_copy(tab_hbm.at[idx2.at[1]], rows_b, sem_b)

    def _half(c, rows_v, w_v, sem):
        pltpu.make_async_copy(tab_hbm.at[idx2.at[c]], rows_v, sem).wait()

        def _scale(g, _):
            vvec = val2[c, pl.ds(g * 16, 16)]
            for t in range(16):
                j = g * 16 + t
                v = vvec[t]
                w_v[j, pl.ds(0, 16)] = rows_v[j, pl.ds(0, 16)] * v
                w_v[j, pl.ds(16, 16)] = rows_v[j, pl.ds(16, 16)] * v
            return 0
        lax.fori_loop(0, CHUNK // 16, _scale, 0)

        @pl.when(c + 2 < NCHUNK)
        def _():
            pltpu.async_copy(tab_hbm.at[idx2.at[c + 2]], rows_v, sem)

        pltpu.sync_copy(w_v, acc_sh.at[rid2.at[c]], add=True)

    def _pair(i, _):
        _half(i * 2, rows_a, w_a, sem_a)
        _half(i * 2 + 1, rows_b, w_b, sem_b)
        return 0
    lax.fori_loop(0, NCHUNK // 2, _pair, 0)
    plsc.subcore_barrier()

    pltpu.sync_copy(acc_sh.at[pl.ds(sid * ROWS_PER_TILE, ROWS_PER_TILE)],
                    out_hbm.at[cid, pl.ds(sid * ROWS_PER_TILE, ROWS_PER_TILE)])


def _sc_lookup(sp_indices, sp_values, sp_row_ids, tab_lin):
    mesh = plsc.VectorSubcoreMesh(core_axis_name="c", subcore_axis_name="s")
    idx_r = sp_indices.reshape(NW, NCHUNK, CHUNK)
    val_r = sp_values.reshape(NW, NCHUNK, CHUNK)
    rid_r = sp_row_ids.reshape(NW, NCHUNK, CHUNK)
    return pl.kernel(
        _sc_body,
        out_type=jax.ShapeDtypeStruct((NC, BATCH, DIM), jnp.float32),
        mesh=mesh,
        compiler_params=pltpu.CompilerParams(use_tc_tiling_on_sc=False),
        scratch_types=[
            pltpu.VMEM((NCHUNK, CHUNK), jnp.int32),
            pltpu.VMEM((NCHUNK, CHUNK), jnp.int32),
            pltpu.VMEM((NCHUNK, CHUNK), jnp.int32),
            pltpu.VMEM((NCHUNK, CHUNK), jnp.float32),
            pltpu.VMEM((CHUNK, DIM), jnp.float32),
            pltpu.VMEM((CHUNK, DIM), jnp.float32),
            pltpu.VMEM((CHUNK, DIM), jnp.float32),
            pltpu.VMEM((CHUNK, DIM), jnp.float32),
            pltpu.VMEM_SHARED((BATCH, DIM), jnp.float32),
            pltpu.SemaphoreType.DMA,
            pltpu.SemaphoreType.DMA,
        ],
    )(idx_r, val_r, rid_r, tab_lin)


def _add_body(p_ref, o_ref):
    o_ref[...] = (p_ref[0] + p_ref[1]).T


@jax.jit
def _run(sp_indices, sp_values, sp_row_ids, tab):
    packed = _transpose_pack(tab)
    tab_lin = packed.reshape(VOCAB_PAD, DIM)
    partials = _sc_lookup(sp_indices, sp_values, sp_row_ids, tab_lin)
    outT = pl.pallas_call(
        _add_body,
        out_shape=jax.ShapeDtypeStruct((DIM, BATCH), jnp.float32),
    )(partials)
    return outT.T


def kernel(sp_indices, sp_values, sp_row_ids, kernel):
    return _run(sp_indices, sp_values, sp_row_ids, kernel)
